# baseline (speedup 1.0000x reference)
import functools

import jax
import jax.numpy as jnp
from jax import lax
from jax.experimental import pallas as pl
from jax.experimental.pallas import tpu as pltpu
from jax.experimental.pallas import tpu_sc as plsc

K = 16
LAMBDA_DECAY = 0.1
LANES = 16


def _tc_body(n_nodes, deg, x_ref, w_ref, ei_ref, ew_ref, h_ref, cn_ref, cw_ref):
    xb = x_ref[...]
    d_in = xb.shape[1]
    decay = jnp.exp(-LAMBDA_DECAY * lax.broadcasted_iota(
        jnp.int32, (1, d_in), 1).astype(jnp.float32))
    h_ref[...] = jnp.dot(xb * decay, w_ref[...], preferred_element_type=jnp.float32)

    bn = cn_ref.shape[0]
    wt = ew_ref[...].T
    dft = ei_ref[1].astype(jnp.float32).T
    node = lax.broadcasted_iota(
        jnp.int32, (1, bn), 1) + pl.program_id(0) * bn
    e_i0 = lax.broadcasted_iota(jnp.int32, (deg, 1), 0)
    valid = node < n_nodes
    spread = ((node * deg + e_i0) % n_nodes).astype(jnp.float32)
    wt = jnp.where(valid, wt, 0.0)
    dft = jnp.where(valid, dft, spread)
    we = wt[:, None, :]
    wd = wt[None, :, :]
    e_i = lax.broadcasted_iota(jnp.int32, (deg, 1, 1), 0)
    d_i = lax.broadcasted_iota(jnp.int32, (1, deg, 1), 1)
    tie = e_i < d_i
    beats = jnp.where((we > wd) | ((we == wd) & tie), 1.0, 0.0)
    rank = jnp.sum(beats, axis=0)
    j_i = lax.broadcasted_iota(jnp.int32, (1, K, 1), 1).astype(jnp.float32)
    ohf = jnp.where(rank[:, None, :] == j_i, 1.0, 0.0)
    cnf = jnp.sum(ohf * dft[:, None, :], axis=0)
    cwk = jnp.sum(ohf * wt[:, None, :], axis=0)
    cn_ref[...] = cnf.T.astype(jnp.int32)
    cw_ref[...] = cwk.T


def _tc_call(x, W, ei3, edge_weight, np_):
    n, d_in = x.shape
    d_out = W.shape[1]
    deg = ei3.shape[2]
    bn = 256
    grid = np_ // bn
    return pl.pallas_call(
        functools.partial(_tc_body, n, deg),
        grid=(grid,),
        in_specs=[
            pl.BlockSpec((bn, d_in), lambda i: (i, 0)),
            pl.BlockSpec((d_in, d_out), lambda i: (0, 0)),
            pl.BlockSpec((2, bn, deg), lambda i: (0, i, 0)),
            pl.BlockSpec((bn, deg), lambda i: (i, 0)),
        ],
        out_specs=[
            pl.BlockSpec((bn, d_out), lambda i: (i, 0)),
            pl.BlockSpec((bn, K), lambda i: (i, 0)),
            pl.BlockSpec((bn, K), lambda i: (i, 0)),
        ],
        out_shape=[
            jax.ShapeDtypeStruct((n, d_out), jnp.float32),
            jax.ShapeDtypeStruct((np_, K), jnp.int32),
            jax.ShapeDtypeStruct((np_, K), jnp.float32),
        ],
    )(x, W, ei3, edge_weight)


def _splat(vec, k):
    idx = jnp.full((LANES, 1), k, dtype=jnp.int32)
    dn = lax.GatherDimensionNumbers(
        offset_dims=(), collapsed_slice_dims=(0,), start_index_map=(0,))
    return lax.gather(vec, idx, dn, slice_sizes=(1,),
                      mode=lax.GatherScatterMode.PROMISE_IN_BOUNDS)


def _sc_call(h, idx_rep, w_rep):
    n, d_out = h.shape
    np_ = w_rep.shape[0]
    info = plsc.get_sparse_core_info()
    nc, ns = info.num_cores, info.num_subcores
    nw = nc * ns
    pt = np_ // nw
    c = 16
    r = c * K
    t = pt // c
    nvec = d_out // LANES
    mesh = plsc.VectorSubcoreMesh(core_axis_name="c", subcore_axis_name="s")
    assert n % c == 0

    @functools.partial(
        pl.kernel,
        mesh=mesh,
        out_type=[
            jax.ShapeDtypeStruct((n, d_out), jnp.float32),
            jax.ShapeDtypeStruct((c, d_out), jnp.float32),
        ],
        scratch_types=[
            pltpu.VMEM((c, K), jnp.int32),
            pltpu.VMEM((c, K), jnp.int32),
            pltpu.VMEM((r,), jnp.int32),
            pltpu.VMEM((r,), jnp.int32),
            pltpu.VMEM((2, r, d_out), jnp.float32),
            pltpu.VMEM((pt, K), jnp.float32),
            pltpu.VMEM((2, c, d_out), jnp.float32),
            pltpu.SemaphoreType.DMA((2,)),
            pltpu.SemaphoreType.DMA((2,)),
            pltpu.SemaphoreType.DMA((2,)),
        ],
    )
    def sc_k(h_hbm, idx_hbm, w_hbm, out_hbm, trash_hbm, ib0, ib1, idx_v0,
             idx_v1, rows_v, w_v, out_v, gsem, osem, isem):
        wid = lax.axis_index("s") * nc + lax.axis_index("c")
        wnode0 = wid * pt
        idx_bufs = (idx_v0, idx_v1)
        ibufs = (ib0, ib1)

        def stage(tt, b):
            pltpu.async_copy(idx_hbm.at[pl.ds(wnode0 + tt * c, c)], ibufs[b],
                             isem.at[b])

        def stage_wait(tt, b):
            pltpu.make_async_copy(idx_hbm.at[pl.ds(wnode0 + tt * c, c)],
                                  ibufs[b], isem.at[b]).wait()

        def fetch(tt, b):
            iv = idx_bufs[b]
            for nn in range(c):
                iv[pl.ds(nn * K, K)] = ibufs[b][nn, :]
            pltpu.async_copy(h_hbm.at[iv], rows_v.at[b], gsem.at[b])

        def put(tt, b):
            ow = wnode0 + tt * c

            @pl.when(ow < n)
            def _():
                pltpu.async_copy(out_v.at[b], out_hbm.at[pl.ds(ow, c)],
                                 osem.at[b])

            @pl.when(ow >= n)
            def _():
                pltpu.async_copy(out_v.at[b], trash_hbm, osem.at[b])

        stage(0, 0)
        stage(1, 1)
        pltpu.sync_copy(w_hbm.at[pl.ds(wnode0, pt)], w_v)
        stage_wait(0, 0)
        fetch(0, 0)

        def pair_body(t2, carry):
            for b in range(2):
                tt = t2 * 2 + b
                ob = 1 - b

                @pl.when(tt + 1 < t)
                def _():
                    stage_wait(tt + 1, ob)
                    fetch(tt + 1, ob)

                pltpu.make_async_copy(
                    h_hbm.at[idx_bufs[b]], rows_v.at[b],
                    gsem.at[b]).wait()

                def node_body(nn, carry2):
                    acc = [None] * nvec
                    wrow = w_v[tt * c + nn, :]
                    for kk in range(K):
                        row = nn * K + kk
                        wsplat = _splat(wrow, kk)
                        for cc in range(nvec):
                            term = wsplat * rows_v[b, row, pl.ds(cc * LANES, LANES)]
                            acc[cc] = term if kk == 0 else acc[cc] + term
                    for cc in range(nvec):
                        out_v[b, nn, pl.ds(cc * LANES, LANES)] = acc[cc]
                    return carry2

                lax.fori_loop(0, c, node_body, 0)

                @pl.when(tt >= 2)
                def _():
                    pltpu.make_async_copy(
                        out_v.at[b],
                        out_hbm.at[pl.ds(wnode0, c)],
                        osem.at[b]).wait()

                put(tt, b)

                @pl.when(tt + 2 < t)
                def _():
                    stage(tt + 2, b)
            return carry

        lax.fori_loop(0, t // 2, pair_body, 0)
        for b in range(2):
            pltpu.make_async_copy(
                out_v.at[b], out_hbm.at[pl.ds(wnode0, c)],
                osem.at[b]).wait()

    return sc_k(h, idx_rep, w_rep)[0]


def kernel(x, edge_index, edge_weight, W, a):
    n, d_in = x.shape
    e = edge_index.shape[1]
    deg = e // n
    nw = 32
    c = 8
    np_ = ((n + nw * c - 1) // (nw * c)) * (nw * c)

    h, cn, cw = _tc_call(x, W, edge_index.reshape(2, n, deg),
                         edge_weight.reshape(n, deg), np_)
    return _sc_call(h, cn, cw)


# speedup vs baseline: 1.0274x; 1.0274x over previous
import functools

import jax
import jax.numpy as jnp
from jax import lax
from jax.experimental import pallas as pl
from jax.experimental.pallas import tpu as pltpu
from jax.experimental.pallas import tpu_sc as plsc

K = 16
LAMBDA_DECAY = 0.1
LANES = 16


def _tc_body(n_nodes, deg, x_ref, w_ref, ei_ref, ew_ref, h_ref, cn_ref, cw_ref):
    xb = x_ref[...]
    d_in = xb.shape[1]
    decay = jnp.exp(-LAMBDA_DECAY * lax.broadcasted_iota(
        jnp.int32, (1, d_in), 1).astype(jnp.float32))
    h_ref[...] = jnp.dot(xb * decay, w_ref[...], preferred_element_type=jnp.float32)

    bn = cn_ref.shape[0]
    wt = ew_ref[...].T
    dft = ei_ref[1].astype(jnp.float32).T
    node = lax.broadcasted_iota(
        jnp.int32, (1, bn), 1) + pl.program_id(0) * bn
    e_i0 = lax.broadcasted_iota(jnp.int32, (deg, 1), 0)
    valid = node < n_nodes
    spread = ((node * deg + e_i0) % n_nodes).astype(jnp.float32)
    wt = jnp.where(valid, wt, 0.0)
    dft = jnp.where(valid, dft, spread)
    we = wt[:, None, :]
    wd = wt[None, :, :]
    e_i = lax.broadcasted_iota(jnp.int32, (deg, 1, 1), 0)
    d_i = lax.broadcasted_iota(jnp.int32, (1, deg, 1), 1)
    tie = e_i < d_i
    beats = jnp.where((we > wd) | ((we == wd) & tie), 1.0, 0.0)
    rank = jnp.sum(beats, axis=0)
    j_i = lax.broadcasted_iota(jnp.int32, (1, K, 1), 1).astype(jnp.float32)
    ohf = jnp.where(rank[:, None, :] == j_i, 1.0, 0.0)
    cnf = jnp.sum(ohf * dft[:, None, :], axis=0)
    cwk = jnp.sum(ohf * wt[:, None, :], axis=0)
    cn_ref[...] = cnf.T.astype(jnp.int32)
    cw_ref[...] = cwk.T


def _tc_call(x, W, ei3, edge_weight, np_):
    n, d_in = x.shape
    d_out = W.shape[1]
    deg = ei3.shape[2]
    bn = 256
    grid = np_ // bn
    return pl.pallas_call(
        functools.partial(_tc_body, n, deg),
        grid=(grid,),
        in_specs=[
            pl.BlockSpec((bn, d_in), lambda i: (i, 0)),
            pl.BlockSpec((d_in, d_out), lambda i: (0, 0)),
            pl.BlockSpec((2, bn, deg), lambda i: (0, i, 0)),
            pl.BlockSpec((bn, deg), lambda i: (i, 0)),
        ],
        out_specs=[
            pl.BlockSpec((bn, d_out), lambda i: (i, 0)),
            pl.BlockSpec((bn, K), lambda i: (i, 0)),
            pl.BlockSpec((bn, K), lambda i: (i, 0)),
        ],
        out_shape=[
            jax.ShapeDtypeStruct((n, d_out), jnp.float32),
            jax.ShapeDtypeStruct((np_, K), jnp.int32),
            jax.ShapeDtypeStruct((np_, K), jnp.float32),
        ],
    )(x, W, ei3, edge_weight)


def _splat(vec, k):
    idx = jnp.full((LANES, 1), k, dtype=jnp.int32)
    dn = lax.GatherDimensionNumbers(
        offset_dims=(), collapsed_slice_dims=(0,), start_index_map=(0,))
    return lax.gather(vec, idx, dn, slice_sizes=(1,),
                      mode=lax.GatherScatterMode.PROMISE_IN_BOUNDS)


def _sc_call(h, idx_rep, w_rep):
    n, d_out = h.shape
    np_ = w_rep.shape[0]
    info = plsc.get_sparse_core_info()
    nc, ns = info.num_cores, info.num_subcores
    nw = nc * ns
    pt = np_ // nw
    c = 16
    r = c * K
    t = pt // c
    nvec = d_out // LANES
    mesh = plsc.VectorSubcoreMesh(core_axis_name="c", subcore_axis_name="s")
    assert n % c == 0

    @functools.partial(
        pl.kernel,
        mesh=mesh,
        out_type=[
            jax.ShapeDtypeStruct((n, d_out), jnp.float32),
            jax.ShapeDtypeStruct((c, d_out), jnp.float32),
        ],
        scratch_types=[
            pltpu.VMEM((pt * K,), jnp.int32),
            pltpu.VMEM((2, r, d_out), jnp.float32),
            pltpu.VMEM((pt, K), jnp.float32),
            pltpu.VMEM((2, c, d_out), jnp.float32),
            pltpu.SemaphoreType.DMA((2,)),
            pltpu.SemaphoreType.DMA((2,)),
        ],
    )
    def sc_k(h_hbm, idx_hbm, w_hbm, out_hbm, trash_hbm, idx_v, rows_v, w_v,
             out_v, gsem, osem):
        wid = lax.axis_index("s") * nc + lax.axis_index("c")
        wnode0 = wid * pt

        def fetch(tt, b):
            pltpu.async_copy(
                h_hbm.at[idx_v.at[pl.ds(tt * r, r)]], rows_v.at[b], gsem.at[b])

        def put(tt, b):
            ow = wnode0 + tt * c

            @pl.when(ow < n)
            def _():
                pltpu.async_copy(out_v.at[b], out_hbm.at[pl.ds(ow, c)],
                                 osem.at[b])

            @pl.when(ow >= n)
            def _():
                pltpu.async_copy(out_v.at[b], trash_hbm, osem.at[b])

        pltpu.sync_copy(idx_hbm.at[pl.ds(wnode0 * K, pt * K)], idx_v)
        pltpu.sync_copy(w_hbm.at[pl.ds(wnode0, pt)], w_v)
        fetch(0, 0)

        def pair_body(t2, carry):
            for b in range(2):
                tt = t2 * 2 + b
                ob = 1 - b

                @pl.when(tt + 1 < t)
                def _():
                    fetch(tt + 1, ob)

                pltpu.make_async_copy(
                    h_hbm.at[idx_v.at[pl.ds(tt * r, r)]], rows_v.at[b],
                    gsem.at[b]).wait()

                def node_body(nn, carry2):
                    acc = [None] * nvec
                    wrow = w_v[tt * c + nn, :]
                    for kk in range(K):
                        row = nn * K + kk
                        wsplat = _splat(wrow, kk)
                        for cc in range(nvec):
                            term = wsplat * rows_v[b, row, pl.ds(cc * LANES, LANES)]
                            acc[cc] = term if kk == 0 else acc[cc] + term
                    for cc in range(nvec):
                        out_v[b, nn, pl.ds(cc * LANES, LANES)] = acc[cc]
                    return carry2

                lax.fori_loop(0, c, node_body, 0)

                @pl.when(tt >= 2)
                def _():
                    pltpu.make_async_copy(
                        out_v.at[b],
                        out_hbm.at[pl.ds(wnode0, c)],
                        osem.at[b]).wait()

                put(tt, b)
            return carry

        lax.fori_loop(0, t // 2, pair_body, 0)
        for b in range(2):
            pltpu.make_async_copy(
                out_v.at[b], out_hbm.at[pl.ds(wnode0, c)],
                osem.at[b]).wait()

    return sc_k(h, idx_rep, w_rep)[0]


def kernel(x, edge_index, edge_weight, W, a):
    n, d_in = x.shape
    e = edge_index.shape[1]
    deg = e // n
    nw = 32
    c = 8
    np_ = ((n + nw * c - 1) // (nw * c)) * (nw * c)

    h, cn, cw = _tc_call(x, W, edge_index.reshape(2, n, deg),
                         edge_weight.reshape(n, deg), np_)
    return _sc_call(h, cn.reshape(-1), cw)


# speedup vs baseline: 1.1051x; 1.0757x over previous
import functools

import jax
import jax.numpy as jnp
from jax import lax
from jax.experimental import pallas as pl
from jax.experimental.pallas import tpu as pltpu
from jax.experimental.pallas import tpu_sc as plsc

K = 16
LAMBDA_DECAY = 0.1
LANES = 16


def _tc_body(n_nodes, deg, x_ref, w_ref, ei_ref, ew_ref, h_ref, cn_ref, cw_ref):
    xb = x_ref[...]
    d_in = xb.shape[1]
    decay = jnp.exp(-LAMBDA_DECAY * lax.broadcasted_iota(
        jnp.int32, (1, d_in), 1).astype(jnp.float32))
    h_ref[...] = jnp.dot(xb * decay, w_ref[...], preferred_element_type=jnp.float32)

    bn = cn_ref.shape[0]
    wt = ew_ref[...].T
    dft = ei_ref[1].astype(jnp.float32).T
    node = lax.broadcasted_iota(
        jnp.int32, (1, bn), 1) + pl.program_id(0) * bn
    e_i0 = lax.broadcasted_iota(jnp.int32, (deg, 1), 0)
    valid = node < n_nodes
    spread = ((node * deg + e_i0) % n_nodes).astype(jnp.float32)
    wt = jnp.where(valid, wt, 0.0)
    dft = jnp.where(valid, dft, spread)
    we = wt[:, None, :]
    wd = wt[None, :, :]
    e_i = lax.broadcasted_iota(jnp.int32, (deg, 1, 1), 0)
    d_i = lax.broadcasted_iota(jnp.int32, (1, deg, 1), 1)
    tie = e_i < d_i
    beats = jnp.where((we > wd) | ((we == wd) & tie), 1.0, 0.0)
    rank = jnp.sum(beats, axis=0)
    j_i = lax.broadcasted_iota(jnp.int32, (1, K, 1), 1).astype(jnp.float32)
    ohf = jnp.where(rank[:, None, :] == j_i, 1.0, 0.0)
    cnf = jnp.sum(ohf * dft[:, None, :], axis=0)
    cwk = jnp.sum(ohf * wt[:, None, :], axis=0)
    cn_ref[...] = cnf.T.astype(jnp.int32)
    cw_ref[...] = cwk.T


def _tc_call(x, W, ei3, edge_weight, np_):
    n, d_in = x.shape
    d_out = W.shape[1]
    deg = ei3.shape[2]
    bn = 512
    grid = np_ // bn
    return pl.pallas_call(
        functools.partial(_tc_body, n, deg),
        grid=(grid,),
        in_specs=[
            pl.BlockSpec((bn, d_in), lambda i: (i, 0)),
            pl.BlockSpec((d_in, d_out), lambda i: (0, 0)),
            pl.BlockSpec((2, bn, deg), lambda i: (0, i, 0)),
            pl.BlockSpec((bn, deg), lambda i: (i, 0)),
        ],
        out_specs=[
            pl.BlockSpec((bn, d_out), lambda i: (i, 0)),
            pl.BlockSpec((bn, K), lambda i: (i, 0)),
            pl.BlockSpec((bn, K), lambda i: (i, 0)),
        ],
        out_shape=[
            jax.ShapeDtypeStruct((n, d_out), jnp.float32),
            jax.ShapeDtypeStruct((np_, K), jnp.int32),
            jax.ShapeDtypeStruct((np_, K), jnp.float32),
        ],
    )(x, W, ei3, edge_weight)


def _splat(vec, k):
    idx = jnp.full((LANES, 1), k, dtype=jnp.int32)
    dn = lax.GatherDimensionNumbers(
        offset_dims=(), collapsed_slice_dims=(0,), start_index_map=(0,))
    return lax.gather(vec, idx, dn, slice_sizes=(1,),
                      mode=lax.GatherScatterMode.PROMISE_IN_BOUNDS)


def _sc_call(h, idx_rep, w_rep):
    n, d_out = h.shape
    np_ = w_rep.shape[0]
    info = plsc.get_sparse_core_info()
    nc, ns = info.num_cores, info.num_subcores
    nw = nc * ns
    pt = np_ // nw
    c = 16
    r = c * K
    t = pt // c
    nvec = d_out // LANES
    mesh = plsc.VectorSubcoreMesh(core_axis_name="c", subcore_axis_name="s")
    assert n % c == 0

    @functools.partial(
        pl.kernel,
        mesh=mesh,
        out_type=[
            jax.ShapeDtypeStruct((n, d_out), jnp.float32),
            jax.ShapeDtypeStruct((c, d_out), jnp.float32),
        ],
        scratch_types=[
            pltpu.VMEM((pt * K,), jnp.int32),
            pltpu.VMEM((2, r, d_out), jnp.float32),
            pltpu.VMEM((pt, K), jnp.float32),
            pltpu.VMEM((2, c, d_out), jnp.float32),
            pltpu.SemaphoreType.DMA((2,)),
            pltpu.SemaphoreType.DMA((2,)),
        ],
    )
    def sc_k(h_hbm, idx_hbm, w_hbm, out_hbm, trash_hbm, idx_v, rows_v, w_v,
             out_v, gsem, osem):
        wid = lax.axis_index("s") * nc + lax.axis_index("c")
        wnode0 = wid * pt

        def fetch(tt, b):
            pltpu.async_copy(
                h_hbm.at[idx_v.at[pl.ds(tt * r, r)]], rows_v.at[b], gsem.at[b])

        def put(tt, b):
            ow = wnode0 + tt * c

            @pl.when(ow < n)
            def _():
                pltpu.async_copy(out_v.at[b], out_hbm.at[pl.ds(ow, c)],
                                 osem.at[b])

            @pl.when(ow >= n)
            def _():
                pltpu.async_copy(out_v.at[b], trash_hbm, osem.at[b])

        pltpu.sync_copy(idx_hbm.at[pl.ds(wnode0 * K, pt * K)], idx_v)
        pltpu.sync_copy(w_hbm.at[pl.ds(wnode0, pt)], w_v)
        fetch(0, 0)

        def pair_body(t2, carry):
            for b in range(2):
                tt = t2 * 2 + b
                ob = 1 - b

                @pl.when(tt + 1 < t)
                def _():
                    fetch(tt + 1, ob)

                pltpu.make_async_copy(
                    h_hbm.at[idx_v.at[pl.ds(tt * r, r)]], rows_v.at[b],
                    gsem.at[b]).wait()

                def node_body(nn, carry2):
                    acc = [None] * nvec
                    wrow = w_v[tt * c + nn, :]
                    for kk in range(K):
                        row = nn * K + kk
                        wsplat = _splat(wrow, kk)
                        for cc in range(nvec):
                            term = wsplat * rows_v[b, row, pl.ds(cc * LANES, LANES)]
                            acc[cc] = term if kk == 0 else acc[cc] + term
                    for cc in range(nvec):
                        out_v[b, nn, pl.ds(cc * LANES, LANES)] = acc[cc]
                    return carry2

                lax.fori_loop(0, c, node_body, 0)

                @pl.when(tt >= 2)
                def _():
                    pltpu.make_async_copy(
                        out_v.at[b],
                        out_hbm.at[pl.ds(wnode0, c)],
                        osem.at[b]).wait()

                put(tt, b)
            return carry

        lax.fori_loop(0, t // 2, pair_body, 0)
        for b in range(2):
            pltpu.make_async_copy(
                out_v.at[b], out_hbm.at[pl.ds(wnode0, c)],
                osem.at[b]).wait()

    return sc_k(h, idx_rep, w_rep)[0]


def kernel(x, edge_index, edge_weight, W, a):
    n, d_in = x.shape
    e = edge_index.shape[1]
    deg = e // n
    nw = 32
    c = 8
    np_ = ((n + nw * c - 1) // (nw * c)) * (nw * c)

    h, cn, cw = _tc_call(x, W, edge_index.reshape(2, n, deg),
                         edge_weight.reshape(n, deg), np_)
    return _sc_call(h, cn.reshape(-1), cw)


# speedup vs baseline: 1.1419x; 1.0332x over previous
import functools

import jax
import jax.numpy as jnp
from jax import lax
from jax.experimental import pallas as pl
from jax.experimental.pallas import tpu as pltpu
from jax.experimental.pallas import tpu_sc as plsc

K = 16
LAMBDA_DECAY = 0.1
LANES = 16


def _tc_body(n_nodes, deg, x_ref, w_ref, ei_ref, ew_ref, h_ref, cn_ref, cw_ref):
    xb = x_ref[...]
    d_in = xb.shape[1]
    decay = jnp.exp(-LAMBDA_DECAY * lax.broadcasted_iota(
        jnp.int32, (1, d_in), 1).astype(jnp.float32))
    h_ref[...] = jnp.dot(xb * decay, w_ref[...], preferred_element_type=jnp.float32)

    bn = cn_ref.shape[0]
    wt = ew_ref[...].T
    dft = ei_ref[1].astype(jnp.float32).T
    node = lax.broadcasted_iota(
        jnp.int32, (1, bn), 1) + pl.program_id(0) * bn
    e_i0 = lax.broadcasted_iota(jnp.int32, (deg, 1), 0)
    valid = node < n_nodes
    spread = ((node * deg + e_i0) % n_nodes).astype(jnp.float32)
    wt = jnp.where(valid, wt, 0.0)
    dft = jnp.where(valid, dft, spread)
    we = wt[:, None, :]
    wd = wt[None, :, :]
    e_i = lax.broadcasted_iota(jnp.int32, (deg, 1, 1), 0)
    d_i = lax.broadcasted_iota(jnp.int32, (1, deg, 1), 1)
    tie = e_i < d_i
    beats = jnp.where((we > wd) | ((we == wd) & tie), 1.0, 0.0)
    rank = jnp.sum(beats, axis=0)
    j_i = lax.broadcasted_iota(jnp.int32, (1, K, 1), 1).astype(jnp.float32)
    ohf = jnp.where(rank[:, None, :] == j_i, 1.0, 0.0)
    cnf = jnp.sum(ohf * dft[:, None, :], axis=0)
    cwk = jnp.sum(ohf * wt[:, None, :], axis=0)
    cn_ref[...] = cnf.T.astype(jnp.int32)
    cw_ref[...] = cwk.T


def _tc_call(x, W, ei3, edge_weight, np_):
    n, d_in = x.shape
    d_out = W.shape[1]
    deg = ei3.shape[2]
    bn = 1024
    grid = np_ // bn
    return pl.pallas_call(
        functools.partial(_tc_body, n, deg),
        grid=(grid,),
        in_specs=[
            pl.BlockSpec((bn, d_in), lambda i: (i, 0)),
            pl.BlockSpec((d_in, d_out), lambda i: (0, 0)),
            pl.BlockSpec((2, bn, deg), lambda i: (0, i, 0)),
            pl.BlockSpec((bn, deg), lambda i: (i, 0)),
        ],
        out_specs=[
            pl.BlockSpec((bn, d_out), lambda i: (i, 0)),
            pl.BlockSpec((bn, K), lambda i: (i, 0)),
            pl.BlockSpec((bn, K), lambda i: (i, 0)),
        ],
        out_shape=[
            jax.ShapeDtypeStruct((n, d_out), jnp.float32),
            jax.ShapeDtypeStruct((np_, K), jnp.int32),
            jax.ShapeDtypeStruct((np_, K), jnp.float32),
        ],
    )(x, W, ei3, edge_weight)


def _splat(vec, k):
    idx = jnp.full((LANES, 1), k, dtype=jnp.int32)
    dn = lax.GatherDimensionNumbers(
        offset_dims=(), collapsed_slice_dims=(0,), start_index_map=(0,))
    return lax.gather(vec, idx, dn, slice_sizes=(1,),
                      mode=lax.GatherScatterMode.PROMISE_IN_BOUNDS)


def _sc_call(h, idx_rep, w_rep):
    n, d_out = h.shape
    np_ = w_rep.shape[0]
    info = plsc.get_sparse_core_info()
    nc, ns = info.num_cores, info.num_subcores
    nw = nc * ns
    pt = np_ // nw
    c = 16
    r = c * K
    t = pt // c
    nvec = d_out // LANES
    mesh = plsc.VectorSubcoreMesh(core_axis_name="c", subcore_axis_name="s")
    assert n % c == 0

    @functools.partial(
        pl.kernel,
        mesh=mesh,
        out_type=[
            jax.ShapeDtypeStruct((n, d_out), jnp.float32),
            jax.ShapeDtypeStruct((c, d_out), jnp.float32),
        ],
        scratch_types=[
            pltpu.VMEM((pt * K,), jnp.int32),
            pltpu.VMEM((2, r, d_out), jnp.float32),
            pltpu.VMEM((pt, K), jnp.float32),
            pltpu.VMEM((2, c, d_out), jnp.float32),
            pltpu.SemaphoreType.DMA((2,)),
            pltpu.SemaphoreType.DMA((2,)),
        ],
    )
    def sc_k(h_hbm, idx_hbm, w_hbm, out_hbm, trash_hbm, idx_v, rows_v, w_v,
             out_v, gsem, osem):
        wid = lax.axis_index("s") * nc + lax.axis_index("c")
        wnode0 = wid * pt

        def fetch(tt, b):
            pltpu.async_copy(
                h_hbm.at[idx_v.at[pl.ds(tt * r, r)]], rows_v.at[b], gsem.at[b])

        def put(tt, b):
            ow = wnode0 + tt * c

            @pl.when(ow < n)
            def _():
                pltpu.async_copy(out_v.at[b], out_hbm.at[pl.ds(ow, c)],
                                 osem.at[b])

            @pl.when(ow >= n)
            def _():
                pltpu.async_copy(out_v.at[b], trash_hbm, osem.at[b])

        pltpu.sync_copy(idx_hbm.at[pl.ds(wnode0 * K, pt * K)], idx_v)
        pltpu.sync_copy(w_hbm.at[pl.ds(wnode0, pt)], w_v)
        fetch(0, 0)

        def pair_body(t2, carry):
            for b in range(2):
                tt = t2 * 2 + b
                ob = 1 - b

                @pl.when(tt + 1 < t)
                def _():
                    fetch(tt + 1, ob)

                pltpu.make_async_copy(
                    h_hbm.at[idx_v.at[pl.ds(tt * r, r)]], rows_v.at[b],
                    gsem.at[b]).wait()

                def node_body(nn, carry2):
                    acc = [None] * nvec
                    wrow = w_v[tt * c + nn, :]
                    for kk in range(K):
                        row = nn * K + kk
                        wsplat = _splat(wrow, kk)
                        for cc in range(nvec):
                            term = wsplat * rows_v[b, row, pl.ds(cc * LANES, LANES)]
                            acc[cc] = term if kk == 0 else acc[cc] + term
                    for cc in range(nvec):
                        out_v[b, nn, pl.ds(cc * LANES, LANES)] = acc[cc]
                    return carry2

                lax.fori_loop(0, c, node_body, 0)

                @pl.when(tt >= 2)
                def _():
                    pltpu.make_async_copy(
                        out_v.at[b],
                        out_hbm.at[pl.ds(wnode0, c)],
                        osem.at[b]).wait()

                put(tt, b)
            return carry

        lax.fori_loop(0, t // 2, pair_body, 0)
        for b in range(2):
            pltpu.make_async_copy(
                out_v.at[b], out_hbm.at[pl.ds(wnode0, c)],
                osem.at[b]).wait()

    return sc_k(h, idx_rep, w_rep)[0]


def kernel(x, edge_index, edge_weight, W, a):
    n, d_in = x.shape
    e = edge_index.shape[1]
    deg = e // n
    nw = 32
    c = 8
    np_ = ((n + nw * c - 1) // (nw * c)) * (nw * c)

    h, cn, cw = _tc_call(x, W, edge_index.reshape(2, n, deg),
                         edge_weight.reshape(n, deg), np_)
    return _sc_call(h, cn.reshape(-1), cw)


# speedup vs baseline: 1.1613x; 1.0170x over previous
import functools

import jax
import jax.numpy as jnp
from jax import lax
from jax.experimental import pallas as pl
from jax.experimental.pallas import tpu as pltpu
from jax.experimental.pallas import tpu_sc as plsc

K = 16
LAMBDA_DECAY = 0.1
LANES = 16


def _tc_body(n_nodes, deg, x_ref, w_ref, ei_ref, ew_ref, h_ref, cn_ref, cw_ref):
    xb = x_ref[...]
    d_in = xb.shape[1]
    decay = jnp.exp(-LAMBDA_DECAY * lax.broadcasted_iota(
        jnp.int32, (1, d_in), 1).astype(jnp.float32))
    h_ref[...] = jnp.dot(xb * decay, w_ref[...], preferred_element_type=jnp.float32)

    bn = cn_ref.shape[0]
    wt = ew_ref[...].T
    dft = ei_ref[1].astype(jnp.float32).T
    node = lax.broadcasted_iota(
        jnp.int32, (1, bn), 1) + pl.program_id(0) * bn
    e_i0 = lax.broadcasted_iota(jnp.int32, (deg, 1), 0)
    valid = node < n_nodes
    spread = ((node * deg + e_i0) % n_nodes).astype(jnp.float32)
    wt = jnp.where(valid, wt, 0.0)
    dft = jnp.where(valid, dft, spread)
    we = wt[:, None, :]
    wd = wt[None, :, :]
    e_i = lax.broadcasted_iota(jnp.int32, (deg, 1, 1), 0)
    d_i = lax.broadcasted_iota(jnp.int32, (1, deg, 1), 1)
    tie = e_i < d_i
    beats = jnp.where((we > wd) | ((we == wd) & tie), 1.0, 0.0)
    rank = jnp.sum(beats, axis=0)
    j_i = lax.broadcasted_iota(jnp.int32, (1, K, 1), 1).astype(jnp.float32)
    ohf = jnp.where(rank[:, None, :] == j_i, 1.0, 0.0)
    cnf = jnp.sum(ohf * dft[:, None, :], axis=0)
    cwk = jnp.sum(ohf * wt[:, None, :], axis=0)
    cn_ref[...] = cnf.T.astype(jnp.int32)
    cw_ref[...] = cwk.T


def _tc_call(x, W, ei3, edge_weight, np_):
    n, d_in = x.shape
    d_out = W.shape[1]
    deg = ei3.shape[2]
    bn = 2048
    grid = np_ // bn
    return pl.pallas_call(
        functools.partial(_tc_body, n, deg),
        grid=(grid,),
        in_specs=[
            pl.BlockSpec((bn, d_in), lambda i: (i, 0)),
            pl.BlockSpec((d_in, d_out), lambda i: (0, 0)),
            pl.BlockSpec((2, bn, deg), lambda i: (0, i, 0)),
            pl.BlockSpec((bn, deg), lambda i: (i, 0)),
        ],
        out_specs=[
            pl.BlockSpec((bn, d_out), lambda i: (i, 0)),
            pl.BlockSpec((bn, K), lambda i: (i, 0)),
            pl.BlockSpec((bn, K), lambda i: (i, 0)),
        ],
        out_shape=[
            jax.ShapeDtypeStruct((n, d_out), jnp.float32),
            jax.ShapeDtypeStruct((np_, K), jnp.int32),
            jax.ShapeDtypeStruct((np_, K), jnp.float32),
        ],
    )(x, W, ei3, edge_weight)


def _splat(vec, k):
    idx = jnp.full((LANES, 1), k, dtype=jnp.int32)
    dn = lax.GatherDimensionNumbers(
        offset_dims=(), collapsed_slice_dims=(0,), start_index_map=(0,))
    return lax.gather(vec, idx, dn, slice_sizes=(1,),
                      mode=lax.GatherScatterMode.PROMISE_IN_BOUNDS)


def _sc_call(h, idx_rep, w_rep):
    n, d_out = h.shape
    np_ = w_rep.shape[0]
    info = plsc.get_sparse_core_info()
    nc, ns = info.num_cores, info.num_subcores
    nw = nc * ns
    pt = np_ // nw
    c = 16
    r = c * K
    t = pt // c
    nvec = d_out // LANES
    mesh = plsc.VectorSubcoreMesh(core_axis_name="c", subcore_axis_name="s")
    assert n % c == 0

    @functools.partial(
        pl.kernel,
        mesh=mesh,
        out_type=[
            jax.ShapeDtypeStruct((n, d_out), jnp.float32),
            jax.ShapeDtypeStruct((c, d_out), jnp.float32),
        ],
        scratch_types=[
            pltpu.VMEM((pt * K,), jnp.int32),
            pltpu.VMEM((2, r, d_out), jnp.float32),
            pltpu.VMEM((pt, K), jnp.float32),
            pltpu.VMEM((2, c, d_out), jnp.float32),
            pltpu.SemaphoreType.DMA((2,)),
            pltpu.SemaphoreType.DMA((2,)),
        ],
    )
    def sc_k(h_hbm, idx_hbm, w_hbm, out_hbm, trash_hbm, idx_v, rows_v, w_v,
             out_v, gsem, osem):
        wid = lax.axis_index("s") * nc + lax.axis_index("c")
        wnode0 = wid * pt

        def fetch(tt, b):
            pltpu.async_copy(
                h_hbm.at[idx_v.at[pl.ds(tt * r, r)]], rows_v.at[b], gsem.at[b])

        def put(tt, b):
            ow = wnode0 + tt * c

            @pl.when(ow < n)
            def _():
                pltpu.async_copy(out_v.at[b], out_hbm.at[pl.ds(ow, c)],
                                 osem.at[b])

            @pl.when(ow >= n)
            def _():
                pltpu.async_copy(out_v.at[b], trash_hbm, osem.at[b])

        pltpu.sync_copy(idx_hbm.at[pl.ds(wnode0 * K, pt * K)], idx_v)
        pltpu.sync_copy(w_hbm.at[pl.ds(wnode0, pt)], w_v)
        fetch(0, 0)

        def pair_body(t2, carry):
            for b in range(2):
                tt = t2 * 2 + b
                ob = 1 - b

                @pl.when(tt + 1 < t)
                def _():
                    fetch(tt + 1, ob)

                pltpu.make_async_copy(
                    h_hbm.at[idx_v.at[pl.ds(tt * r, r)]], rows_v.at[b],
                    gsem.at[b]).wait()

                def node_body(nn, carry2):
                    acc = [None] * nvec
                    wrow = w_v[tt * c + nn, :]
                    for kk in range(K):
                        row = nn * K + kk
                        wsplat = _splat(wrow, kk)
                        for cc in range(nvec):
                            term = wsplat * rows_v[b, row, pl.ds(cc * LANES, LANES)]
                            acc[cc] = term if kk == 0 else acc[cc] + term
                    for cc in range(nvec):
                        out_v[b, nn, pl.ds(cc * LANES, LANES)] = acc[cc]
                    return carry2

                lax.fori_loop(0, c, node_body, 0)

                @pl.when(tt >= 2)
                def _():
                    pltpu.make_async_copy(
                        out_v.at[b],
                        out_hbm.at[pl.ds(wnode0, c)],
                        osem.at[b]).wait()

                put(tt, b)
            return carry

        lax.fori_loop(0, t // 2, pair_body, 0)
        for b in range(2):
            pltpu.make_async_copy(
                out_v.at[b], out_hbm.at[pl.ds(wnode0, c)],
                osem.at[b]).wait()

    return sc_k(h, idx_rep, w_rep)[0]


def kernel(x, edge_index, edge_weight, W, a):
    n, d_in = x.shape
    e = edge_index.shape[1]
    deg = e // n
    nw = 32
    c = 8
    np_ = ((n + nw * c - 1) // (nw * c)) * (nw * c)

    h, cn, cw = _tc_call(x, W, edge_index.reshape(2, n, deg),
                         edge_weight.reshape(n, deg), np_)
    return _sc_call(h, cn.reshape(-1), cw)
